# Initial kernel scaffold; baseline (speedup 1.0000x reference)
import jax
import jax.numpy as jnp
from jax.experimental import pallas as pl


def kernel(protein_pos, ligand_pos, batch_protein, batch_ligand):
    raise NotImplementedError("write your pallas kernel here")


# speedup vs baseline: 3.6644x; 3.6644x over previous
import functools

import jax
import jax.numpy as jnp
from jax import lax
from jax.experimental import pallas as pl
from jax.experimental.pallas import tpu as pltpu
from jax.experimental.pallas import tpu_sc as plsc

NP = 80000
NL = 20000
G = 256

NC, NS = 2, 16
P1 = NP // NS
P2 = 2512
P2_LAST = NP - P2
L2 = 640
L2_LAST = NL - L2


def _sc_body(pp, lp, bp, bl, po, lo, off_out,
             pos1_v, bat1_v, acc_v, part_v, big_v, tmp16_v,
             off48_v, offt_v, pos2_v, bat2_v, out2_v, lpos_v, lbat_v,
             lout_v, shp_all):
    c = lax.axis_index("c")
    s = lax.axis_index("s")
    w = s * NC + c

    zeros16 = jnp.zeros((16,), jnp.float32)

    lane = lax.iota(jnp.int32, 16)

    pltpu.sync_copy(pp.at[pl.ds(s * (P1 * 3), P1 * 3)], pos1_v)
    pltpu.sync_copy(bp.at[pl.ds(s * P1, P1)], bat1_v)

    @pl.loop(0, 256, unroll=8)
    def zero_acc(k):
        acc_v[pl.ds(k * 16, 16)] = zeros16

    @pl.loop(0, P1 // 4, unroll=8)
    def accum(j):
        ln = lax.iota(jnp.int32, 16)
        ro = lax.div(ln, 4)
        c4 = lax.rem(ln, 4)
        ctf = lax.div(c4, 3).astype(jnp.float32)
        rb = j * 4
        b = plsc.load_gather(
            bat1_v, [jnp.full((16,), rb, jnp.int32) + ro])
        g = plsc.load_gather(
            pos1_v, [jnp.full((16,), rb * 3, jnp.int32)
                     + ro * 3 + jnp.minimum(c4, 2)])
        p = g * (1.0 - ctf) + ctf
        plsc.addupdate_scatter(acc_v, [b * 4 + ro * 1024 + c4], p)

    for k in range(64):
        part_v[pl.ds(16 * k, 16)] = (
            acc_v[pl.ds(16 * k, 16)]
            + acc_v[pl.ds(1024 + 16 * k, 16)]
            + acc_v[pl.ds(2048 + 16 * k, 16)]
            + acc_v[pl.ds(3072 + 16 * k, 16)])

    pltpu.sync_copy(part_v, shp_all.at[pl.ds(s * 1024, 1024)])
    plsc.subcore_barrier()
    pltpu.sync_copy(shp_all, big_v)
    cidx = lane | 3
    for r in range(64):
        vsum = big_v[pl.ds(16 * r, 16)]
        for t in range(1, 16):
            vsum = vsum + big_v[pl.ds(t * 1024 + 16 * r, 16)]
        tmp16_v[...] = vsum
        cnt = plsc.load_gather(tmp16_v, [cidx])
        offt_v[pl.ds(16 * r, 16)] = vsum / jnp.maximum(cnt, 1.0)

    @pl.when((c == 0) & (s == 0))
    def _():
        for k in range(48):
            f = 16 * k + lane
            gg = lax.div(f, 3)
            cc = f - gg * 3
            off48_v[pl.ds(16 * k, 16)] = plsc.load_gather(
                offt_v, [gg * 4 + cc])
        pltpu.sync_copy(off48_v, off_out)

    def subtract(bat_ref, pos_ref, out_ref, k):
        ln = lax.iota(jnp.int32, 16)
        fb = jnp.full((16,), k * 16, jnp.int32) + ln
        r = lax.div(fb, 3)
        c3 = fb - r * 3
        b = plsc.load_gather(bat_ref, [r])
        o = plsc.load_gather(offt_v, [b * 4 + c3])
        out_ref[pl.ds(k * 16, 16)] = pos_ref[pl.ds(k * 16, 16)] - o

    start2 = jnp.minimum(w * P2, P2_LAST)
    pltpu.sync_copy(pp.at[pl.ds(start2 * 3, P2 * 3)], pos2_v)
    pltpu.sync_copy(bp.at[pl.ds(start2, P2)], bat2_v)

    @plsc.parallel_loop(0, P2 * 3 // 16, unroll=8)
    def body_p(k):
        subtract(bat2_v, pos2_v, out2_v, k)
    pltpu.sync_copy(out2_v, po.at[pl.ds(start2 * 3, P2 * 3)])

    startl = jnp.minimum(w * L2, L2_LAST)
    pltpu.sync_copy(lp.at[pl.ds(startl * 3, L2 * 3)], lpos_v)
    pltpu.sync_copy(bl.at[pl.ds(startl, L2)], lbat_v)

    @plsc.parallel_loop(0, L2 * 3 // 16, unroll=8)
    def body_l(k):
        subtract(lbat_v, lpos_v, lout_v, k)
    pltpu.sync_copy(lout_v, lo.at[pl.ds(startl * 3, L2 * 3)])


@functools.partial(
    pl.kernel,
    out_type=(
        jax.ShapeDtypeStruct((NP * 3,), jnp.float32),
        jax.ShapeDtypeStruct((NL * 3,), jnp.float32),
        jax.ShapeDtypeStruct((G * 3,), jnp.float32),
    ),
    mesh=plsc.VectorSubcoreMesh(
        core_axis_name="c", subcore_axis_name="s",
        num_cores=NC, num_subcores=NS),
    compiler_params=pltpu.CompilerParams(
        needs_layout_passes=False, skip_device_barrier=True),
    scratch_types=[
        pltpu.VMEM((P1 * 3,), jnp.float32),
        pltpu.VMEM((P1,), jnp.int32),
        pltpu.VMEM((4096,), jnp.float32),
        pltpu.VMEM((1024,), jnp.float32),
        pltpu.VMEM((16384,), jnp.float32),
        pltpu.VMEM((16,), jnp.float32),
        pltpu.VMEM((G * 3,), jnp.float32),
        pltpu.VMEM((G * 4,), jnp.float32),
        pltpu.VMEM((P2 * 3,), jnp.float32),
        pltpu.VMEM((P2,), jnp.int32),
        pltpu.VMEM((P2 * 3,), jnp.float32),
        pltpu.VMEM((L2 * 3,), jnp.float32),
        pltpu.VMEM((L2,), jnp.int32),
        pltpu.VMEM((L2 * 3,), jnp.float32),
        pltpu.VMEM_SHARED((16384,), jnp.float32),
    ],
)
def _center_pos_sc(pp, lp, bp, bl, po, lo, off_out, *scratch):
    _sc_body(pp, lp, bp, bl, po, lo, off_out, *scratch)


def kernel(protein_pos, ligand_pos, batch_protein, batch_ligand):
    po, lo, off = _center_pos_sc(
        protein_pos.reshape(-1),
        ligand_pos.reshape(-1),
        batch_protein.astype(jnp.int32),
        batch_ligand.astype(jnp.int32),
    )
    return (po.reshape(NP, 3), lo.reshape(NL, 3), off.reshape(G, 3))
